# baseline (speedup 1.0000x reference)
import jax
import jax.numpy as jnp
from jax.experimental import pallas as pl

N = 1024
K = 128
NB = N - K
NA = N - 8
KA = K - 8
BR = 256


def _iota2(shape, dim):
    return jax.lax.broadcasted_iota(jnp.int32, shape, dim)


def _constants(o):
    f32 = jnp.float32
    o128 = o.reshape(K, 16)
    x = jnp.concatenate([o128] * 8, axis=1)
    b = jnp.where((_iota2((K, K), 0) // 16) == (_iota2((K, K), 1) // 16),
                  x, 0.0)
    gp, gq = _iota2((KA, K), 0), _iota2((KA, K), 1)
    g = ((gp // 15) * 16 + (gp % 15) + 1 == gq).astype(f32)
    e = (_iota2((8, K), 0) * 16 == _iota2((8, K), 1)).astype(f32)
    return b, g, e


def _main_kernel(a_ref, apeek_ref, o_ref,
                 d_ref, ar_ref, fc_ref, right_ref, fw_ref, mc_ref, mw_ref):
    f32 = jnp.float32
    i = pl.program_id(0)
    dot = lambda u, v: jnp.dot(u, v, preferred_element_type=f32)
    b, g, e = _constants(o_ref[...])
    ablk = a_ref[...]
    apeek = apeek_ref[...]
    arows_l = jnp.concatenate([ablk[8:, :K], apeek[:, :K]], axis=0)

    fcl = dot(dot(arows_l, b.T), g.T)

    rowid = BR * i + _iota2((BR, N), 0)
    colid = _iota2((BR, N), 1)

    @pl.when(i == 0)
    def _():
        a_top = ablk[:K, :]
        t = dot(b, a_top)
        m = dot(t[:, :K], b.T)
        ri, ci = _iota2((K, N), 0), _iota2((K, N), 1)
        act_r = (ri % 16) != 0
        act_c = (ci >= K) | ((ci % 16) != 0)
        af_top = jnp.concatenate([m, t[:, K:]], axis=1)
        d_top = jnp.where((ri == ci) | (act_r & act_c), af_top, 0.0)
        s = dot(b.T, d_top)
        d_ref[:K, :] = d_top
        ar_ref[:K, :K] = dot(s[:, :K], b)
        ar_ref[:K, K:] = s[:, K:]

        t_bot = dot(ablk[K:, :K], b.T)
        cmask = (_iota2((K, K), 1) % 16) != 0
        d_bl = jnp.where(cmask, t_bot, 0.0)
        d_ref[K:, :K] = d_bl
        d_ref[K:, K:] = ablk[K:, K:]
        ar_ref[K:, :K] = dot(d_bl, b)
        ar_ref[K:, K:] = ablk[K:, K:]

        fc_ref[:KA, :KA] = dot(dot(g, m), g.T)
        fc_ref[:KA, KA:] = dot(g, t[:, K:])
        fc_ref[KA:, :KA] = fcl[KA:]
        fc_ref[KA:, KA:] = jnp.concatenate(
            [ablk[K:, K:], apeek[:, K:]], axis=0)

        right_ref[:K, :K] = b
        right_ref[:K, K:] = jnp.zeros((K, NB), f32)
        right_ref[K:, :] = (rowid[K:] == colid[K:]).astype(f32)
        fw_ref[:KA, :K] = dot(g, b)
        fw_ref[:KA, K:] = jnp.zeros((KA, NB), f32)
        fw_ref[KA:, :] = (colid[:136] == _iota2((136, N), 0) + K).astype(f32)

        eme = dot(dot(e, m), e.T)
        mc_ref[...] = jnp.where(_iota2((8, 8), 0) == _iota2((8, 8), 1),
                                eme, 0.0)
        mw_ref[:, :K] = dot(e, b)
        mw_ref[:, K:] = jnp.zeros((8, NB), f32)

    @pl.when(i > 0)
    def _():
        t_blk = dot(ablk[:, :K], b.T)
        cmask = (_iota2((BR, K), 1) % 16) != 0
        d_l = jnp.where(cmask, t_blk, 0.0)
        d_ref[:, :K] = d_l
        d_ref[:, K:] = ablk[:, K:]
        ar_ref[:, :K] = dot(d_l, b)
        ar_ref[:, K:] = ablk[:, K:]
        fc_ref[:, :KA] = fcl
        fc_ref[:BR - 8, KA:] = ablk[8:, K:]
        fc_ref[BR - 8:, KA:] = apeek[:, K:]
        right_ref[...] = (rowid == colid).astype(f32)
        fw_ref[...] = (colid == rowid + 8).astype(f32)


def kernel(A_dense, O, rot_rows, rot_cols, wav_idx, act_idx):
    f32 = jnp.float32
    sds = jax.ShapeDtypeStruct

    d, a_rec, fc, right, fw, mc, mw = pl.pallas_call(
        _main_kernel,
        grid=(4,),
        in_specs=[
            pl.BlockSpec((BR, N), lambda i: (i, 0)),
            pl.BlockSpec((8, N),
                         lambda i: (jnp.minimum(32 * (i + 1), 127), 0)),
            pl.BlockSpec((8, 16, 16), lambda i: (0, 0, 0)),
        ],
        out_specs=[
            pl.BlockSpec((BR, N), lambda i: (i, 0)),
            pl.BlockSpec((BR, N), lambda i: (i, 0)),
            pl.BlockSpec((BR, NA), lambda i: (i, 0)),
            pl.BlockSpec((BR, N), lambda i: (i, 0)),
            pl.BlockSpec((BR, N), lambda i: (i, 0)),
            pl.BlockSpec((8, 8), lambda i: (0, 0)),
            pl.BlockSpec((8, N), lambda i: (0, 0)),
        ],
        out_shape=[
            sds((N, N), f32),
            sds((N, N), f32),
            sds((NA, NA), f32),
            sds((N, N), f32),
            sds((NA, N), f32),
            sds((8, 8), f32),
            sds((8, N), f32),
        ],
    )(A_dense, A_dense, O)

    return (a_rec, right, d, mc, fc, mw, fw)


# speedup vs baseline: 1.0516x; 1.0516x over previous
import jax
import jax.numpy as jnp
from jax.experimental import pallas as pl

N = 1024
K = 128
NB = N - K
NA = N - 8
KA = K - 8
BR = 512


def _iota2(shape, dim):
    return jax.lax.broadcasted_iota(jnp.int32, shape, dim)


def _constants(o):
    f32 = jnp.float32
    o128 = o.reshape(K, 16)
    x = jnp.concatenate([o128] * 8, axis=1)
    b = jnp.where((_iota2((K, K), 0) // 16) == (_iota2((K, K), 1) // 16),
                  x, 0.0)
    gp, gq = _iota2((KA, K), 0), _iota2((KA, K), 1)
    g = ((gp // 15) * 16 + (gp % 15) + 1 == gq).astype(f32)
    e = (_iota2((8, K), 0) * 16 == _iota2((8, K), 1)).astype(f32)
    return b, g, e


def _main_kernel(a_ref, apeek_ref, o_ref,
                 d_ref, ar_ref, fc_ref, right_ref, fw_ref, mc_ref, mw_ref):
    f32 = jnp.float32
    i = pl.program_id(0)
    dot = lambda u, v: jnp.dot(u, v, preferred_element_type=f32)
    b, g, e = _constants(o_ref[...])
    ablk = a_ref[...]
    apeek = apeek_ref[...]
    arows_l = jnp.concatenate([ablk[8:, :K], apeek[:, :K]], axis=0)

    fcl = dot(dot(arows_l, b.T), g.T)

    rowid = BR * i + _iota2((BR, N), 0)
    colid = _iota2((BR, N), 1)

    @pl.when(i == 0)
    def _():
        a_top = ablk[:K, :]
        t = dot(b, a_top)
        m = dot(t[:, :K], b.T)
        ri, ci = _iota2((K, N), 0), _iota2((K, N), 1)
        act_r = (ri % 16) != 0
        act_c = (ci >= K) | ((ci % 16) != 0)
        af_top = jnp.concatenate([m, t[:, K:]], axis=1)
        d_top = jnp.where((ri == ci) | (act_r & act_c), af_top, 0.0)
        s = dot(b.T, d_top)
        d_ref[:K, :] = d_top
        ar_ref[:K, :K] = dot(s[:, :K], b)
        ar_ref[:K, K:] = s[:, K:]

        t_bot = dot(ablk[K:, :K], b.T)
        cmask = (_iota2((BR - K, K), 1) % 16) != 0
        d_bl = jnp.where(cmask, t_bot, 0.0)
        d_ref[K:, :K] = d_bl
        d_ref[K:, K:] = ablk[K:, K:]
        ar_ref[K:, :K] = dot(d_bl, b)
        ar_ref[K:, K:] = ablk[K:, K:]

        fc_ref[:KA, :KA] = dot(dot(g, m), g.T)
        fc_ref[:KA, KA:] = dot(g, t[:, K:])
        fc_ref[KA:, :KA] = fcl[KA:]
        fc_ref[KA:, KA:] = jnp.concatenate(
            [ablk[K:, K:], apeek[:, K:]], axis=0)

        right_ref[:K, :K] = b
        right_ref[:K, K:] = jnp.zeros((K, NB), f32)
        right_ref[K:, :] = (rowid[K:] == colid[K:]).astype(f32)
        fw_ref[:KA, :K] = dot(g, b)
        fw_ref[:KA, K:] = jnp.zeros((KA, NB), f32)
        fw_ref[KA:, :] = (
            colid[:BR - KA] == _iota2((BR - KA, N), 0) + K).astype(f32)

        eme = dot(dot(e, m), e.T)
        mc_ref[...] = jnp.where(_iota2((8, 8), 0) == _iota2((8, 8), 1),
                                eme, 0.0)
        mw_ref[:, :K] = dot(e, b)
        mw_ref[:, K:] = jnp.zeros((8, NB), f32)

    @pl.when(i > 0)
    def _():
        t_blk = dot(ablk[:, :K], b.T)
        cmask = (_iota2((BR, K), 1) % 16) != 0
        d_l = jnp.where(cmask, t_blk, 0.0)
        d_ref[:, :K] = d_l
        d_ref[:, K:] = ablk[:, K:]
        ar_ref[:, :K] = dot(d_l, b)
        ar_ref[:, K:] = ablk[:, K:]
        fc_ref[:, :KA] = fcl
        fc_ref[:BR - 8, KA:] = ablk[8:, K:]
        fc_ref[BR - 8:, KA:] = apeek[:, K:]
        right_ref[...] = (rowid == colid).astype(f32)
        fw_ref[...] = (colid == rowid + 8).astype(f32)


def kernel(A_dense, O, rot_rows, rot_cols, wav_idx, act_idx):
    f32 = jnp.float32
    sds = jax.ShapeDtypeStruct

    d, a_rec, fc, right, fw, mc, mw = pl.pallas_call(
        _main_kernel,
        grid=(N // BR,),
        in_specs=[
            pl.BlockSpec((BR, N), lambda i: (i, 0)),
            pl.BlockSpec((8, N),
                         lambda i: (jnp.minimum(BR // 8 * (i + 1), 127), 0)),
            pl.BlockSpec((8, 16, 16), lambda i: (0, 0, 0)),
        ],
        out_specs=[
            pl.BlockSpec((BR, N), lambda i: (i, 0)),
            pl.BlockSpec((BR, N), lambda i: (i, 0)),
            pl.BlockSpec((BR, NA), lambda i: (i, 0)),
            pl.BlockSpec((BR, N), lambda i: (i, 0)),
            pl.BlockSpec((BR, N), lambda i: (i, 0)),
            pl.BlockSpec((8, 8), lambda i: (0, 0)),
            pl.BlockSpec((8, N), lambda i: (0, 0)),
        ],
        out_shape=[
            sds((N, N), f32),
            sds((N, N), f32),
            sds((NA, NA), f32),
            sds((N, N), f32),
            sds((NA, N), f32),
            sds((8, 8), f32),
            sds((8, N), f32),
        ],
    )(A_dense, A_dense, O)

    return (a_rec, right, d, mc, fc, mw, fw)
